# baseline (speedup 1.0000x reference)
import functools
import jax
import jax.numpy as jnp
from jax import lax
from jax.experimental import pallas as pl
from jax.experimental.pallas import tpu as pltpu
from jax.experimental.pallas import tpu_sc as plsc

_BT = 256
_SC_COLS = 4096
_NBW = 128
_DCH = 80
_NG = _NBW // 16
_C = 5
_D = 10000
_TC_COLS = 16384 - _SC_COLS


def _tc_body(w_ref, xt_ref, o_ref):
    o_ref[...] = jax.lax.dot_general(
        w_ref[...], xt_ref[...],
        dimension_numbers=(((1,), (0,)), ((), ())),
        preferred_element_type=jnp.float32,
    )


def _sc_kernel(xt_hbm, w_hbm, o_hbm, wbuf, xbuf, obuf, sem):
    wid = lax.axis_index("s") * 2 + lax.axis_index("c")
    base = wid * _NBW
    src_base = _TC_COLS + base

    pltpu.sync_copy(w_hbm, wbuf)

    def chunk_body(g, acc):
        d0 = g * _DCH
        pltpu.async_copy(
            xt_hbm.at[pl.ds(d0, _DCH), pl.ds(src_base, _NBW)], xbuf, sem
        ).wait()

        def d_body(t, acc):
            wvs = [wbuf[c, pl.ds(d0 + t * 16, 16)] for c in range(_C)]
            out = list(acc)
            for k in range(16):
                ws = [wvs[c][k] for c in range(_C)]
                for gi in range(_NG):
                    xv = xbuf[t * 16 + k, pl.ds(gi * 16, 16)]
                    for c in range(_C):
                        out[gi * _C + c] = out[gi * _C + c] + xv * ws[c]
            return tuple(out)

        return lax.fori_loop(0, _DCH // 16, d_body, acc)

    acc0 = tuple(jnp.zeros((16,), jnp.float32) for _ in range(_NG * _C))
    acc = lax.fori_loop(0, _D // _DCH, chunk_body, acc0)

    for gi in range(_NG):
        for c in range(_C):
            obuf[c, pl.ds(gi * 16, 16)] = acc[gi * _C + c]
    pltpu.sync_copy(obuf, o_hbm.at[:, pl.ds(base, _NBW)])


def _sc_matmul(xt, weight):
    mesh = plsc.VectorSubcoreMesh(core_axis_name="c", subcore_axis_name="s")
    k = functools.partial(
        pl.kernel,
        out_type=jax.ShapeDtypeStruct((_C, _SC_COLS), jnp.float32),
        mesh=mesh,
        scratch_types=[
            pltpu.VMEM((_C, _D), jnp.float32),
            pltpu.VMEM((_DCH, _NBW), jnp.float32),
            pltpu.VMEM((_C, _NBW), jnp.float32),
            pltpu.SemaphoreType.DMA,
        ],
    )(_sc_kernel)
    return k(xt, weight)


def kernel(x, weight):
    B, D = x.shape
    C = weight.shape[0]
    tc_cols = B - _SC_COLS
    xt = x.T
    tc_out = pl.pallas_call(
        _tc_body,
        grid=(tc_cols // _BT,),
        in_specs=[
            pl.BlockSpec((C, D), lambda j: (0, 0)),
            pl.BlockSpec((D, _BT), lambda j: (0, j)),
        ],
        out_specs=pl.BlockSpec((C, _BT), lambda j: (0, j)),
        out_shape=jax.ShapeDtypeStruct((C, tc_cols), jnp.float32),
    )(weight, xt)
    sc_out = _sc_matmul(xt, weight)
    out = jnp.concatenate([tc_out, sc_out], axis=1)
    return out.T


# speedup vs baseline: 1.0812x; 1.0812x over previous
import functools
import jax
import jax.numpy as jnp
from jax import lax
from jax.experimental import pallas as pl
from jax.experimental.pallas import tpu as pltpu
from jax.experimental.pallas import tpu_sc as plsc

_BT = 256
_SC_COLS = 4096
_NBW = 128
_DCH = 80
_NG = _NBW // 16
_C = 5
_D = 10000
_TC_COLS = 16384 - _SC_COLS
_SC_D = 4800
_RB = 400


def _tc_body(w_ref, xt_ref, o_ref):
    o_ref[...] = jax.lax.dot_general(
        w_ref[...], xt_ref[...],
        dimension_numbers=(((1,), (0,)), ((), ())),
        preferred_element_type=jnp.float32,
    )


def _sc_kernel(xt_hbm, w_hbm, o_hbm, wbuf, xbuf, obuf, sem):
    wid = lax.axis_index("s") * 2 + lax.axis_index("c")
    base = wid * _NBW
    src_base = _TC_COLS + base

    pltpu.sync_copy(w_hbm, wbuf)

    def chunk_body(g, acc):
        d0 = g * _DCH
        pltpu.async_copy(
            xt_hbm.at[pl.ds(d0, _DCH), pl.ds(src_base, _NBW)], xbuf, sem
        ).wait()

        def d_body(t, acc):
            wvs = [wbuf[c, pl.ds(d0 + t * 16, 16)] for c in range(_C)]
            out = list(acc)
            for k in range(16):
                ws = [wvs[c][k] for c in range(_C)]
                for gi in range(_NG):
                    xv = xbuf[t * 16 + k, pl.ds(gi * 16, 16)]
                    for c in range(_C):
                        out[gi * _C + c] = out[gi * _C + c] + xv * ws[c]
            return tuple(out)

        return lax.fori_loop(0, _DCH // 16, d_body, acc)

    acc0 = tuple(jnp.zeros((16,), jnp.float32) for _ in range(_NG * _C))
    acc = lax.fori_loop(0, _SC_D // _DCH, chunk_body, acc0)

    for gi in range(_NG):
        for c in range(_C):
            obuf[c, pl.ds(gi * 16, 16)] = acc[gi * _C + c]
    pltpu.sync_copy(obuf, o_hbm.at[:, pl.ds(base, _NBW)])


def _sc_matmul(xt, weight):
    mesh = plsc.VectorSubcoreMesh(core_axis_name="c", subcore_axis_name="s")
    k = functools.partial(
        pl.kernel,
        out_type=jax.ShapeDtypeStruct((_C, _SC_COLS), jnp.float32),
        mesh=mesh,
        scratch_types=[
            pltpu.VMEM((_C, _D), jnp.float32),
            pltpu.VMEM((_DCH, _NBW), jnp.float32),
            pltpu.VMEM((_C, _NBW), jnp.float32),
            pltpu.SemaphoreType.DMA,
        ],
    )(_sc_kernel)
    return k(xt, weight)


def _tc2_body(wt_ref, xt_ref, o_ref):
    @pl.when(pl.program_id(1) == 0)
    def _init():
        o_ref[...] = jnp.zeros_like(o_ref)

    o_ref[...] += jax.lax.dot_general(
        wt_ref[...], xt_ref[...],
        dimension_numbers=(((0,), (0,)), ((), ())),
        preferred_element_type=jnp.float32,
    )


def kernel(x, weight):
    B, D = x.shape
    C = weight.shape[0]
    xt = x.T
    tc_out = pl.pallas_call(
        _tc_body,
        grid=(_TC_COLS // _BT,),
        in_specs=[
            pl.BlockSpec((C, D), lambda j: (0, 0)),
            pl.BlockSpec((D, _BT), lambda j: (0, j)),
        ],
        out_specs=pl.BlockSpec((C, _BT), lambda j: (0, j)),
        out_shape=jax.ShapeDtypeStruct((C, _TC_COLS), jnp.float32),
    )(weight, xt)
    rb0 = _SC_D // _RB
    tc2_out = pl.pallas_call(
        _tc2_body,
        grid=(_SC_COLS // _BT, (_D - _SC_D) // _RB),
        in_specs=[
            pl.BlockSpec((_RB, C), lambda j, i: (rb0 + i, 0)),
            pl.BlockSpec((_RB, _BT), lambda j, i: (rb0 + i, _TC_COLS // _BT + j)),
        ],
        out_specs=pl.BlockSpec((C, _BT), lambda j, i: (0, j)),
        out_shape=jax.ShapeDtypeStruct((C, _SC_COLS), jnp.float32),
    )(weight.T, xt)
    sc_out = _sc_matmul(xt, weight)
    out = jnp.concatenate([tc_out, sc_out + tc2_out], axis=1)
    return out.T


# speedup vs baseline: 1.1977x; 1.1078x over previous
import functools
import jax
import jax.numpy as jnp
from jax import lax
from jax.experimental import pallas as pl
from jax.experimental.pallas import tpu as pltpu
from jax.experimental.pallas import tpu_sc as plsc

_BT = 256
_SC_COLS = 4096
_NBW = 128
_DCH = 160
_NG = _NBW // 16
_C = 5
_D = 10000
_TC_COLS = 16384 - _SC_COLS
_SC_D = 6400
_RB = 400


def _tc_body(w_ref, xt_ref, o_ref):
    o_ref[...] = jax.lax.dot_general(
        w_ref[...], xt_ref[...],
        dimension_numbers=(((1,), (0,)), ((), ())),
        preferred_element_type=jnp.float32,
    )


def _sc_kernel(xt_hbm, w_hbm, o_hbm, wbuf, xbuf0, xbuf1, obuf, sem0, sem1):
    wid = lax.axis_index("s") * 2 + lax.axis_index("c")
    base = wid * _NBW
    src_base = _TC_COLS + base
    n_chunks = _SC_D // _DCH

    pltpu.sync_copy(w_hbm, wbuf)

    def copy_chunk(g, buf, sem):
        return pltpu.make_async_copy(
            xt_hbm.at[pl.ds(g * _DCH, _DCH), pl.ds(src_base, _NBW)], buf, sem
        )

    def consume(g, xbuf, acc):
        d0 = g * _DCH

        def d_body(t, acc):
            wvs = [wbuf[c, pl.ds(d0 + t * 16, 16)] for c in range(_C)]
            out = list(acc)
            for k in range(16):
                ws = [wvs[c][k] for c in range(_C)]
                for gi in range(_NG):
                    xv = xbuf[t * 16 + k, pl.ds(gi * 16, 16)]
                    for c in range(_C):
                        out[gi * _C + c] = out[gi * _C + c] + xv * ws[c]
            return tuple(out)

        return lax.fori_loop(0, _DCH // 16, d_body, acc)

    copy_chunk(0, xbuf0, sem0).start()

    def pair_body(t2, acc):
        g0 = 2 * t2
        copy_chunk(g0 + 1, xbuf1, sem1).start()
        copy_chunk(g0, xbuf0, sem0).wait()
        acc = consume(g0, xbuf0, acc)

        @pl.when(g0 + 2 < n_chunks)
        def _next():
            copy_chunk(g0 + 2, xbuf0, sem0).start()

        copy_chunk(g0 + 1, xbuf1, sem1).wait()
        return consume(g0 + 1, xbuf1, acc)

    acc0 = tuple(jnp.zeros((16,), jnp.float32) for _ in range(_NG * _C))
    acc = lax.fori_loop(0, n_chunks // 2, pair_body, acc0)

    for gi in range(_NG):
        for c in range(_C):
            obuf[c, pl.ds(gi * 16, 16)] = acc[gi * _C + c]
    pltpu.sync_copy(obuf, o_hbm.at[:, pl.ds(base, _NBW)])


def _sc_matmul(xt, weight):
    mesh = plsc.VectorSubcoreMesh(core_axis_name="c", subcore_axis_name="s")
    k = functools.partial(
        pl.kernel,
        out_type=jax.ShapeDtypeStruct((_C, _SC_COLS), jnp.float32),
        mesh=mesh,
        scratch_types=[
            pltpu.VMEM((_C, _D), jnp.float32),
            pltpu.VMEM((_DCH, _NBW), jnp.float32),
            pltpu.VMEM((_DCH, _NBW), jnp.float32),
            pltpu.VMEM((_C, _NBW), jnp.float32),
            pltpu.SemaphoreType.DMA,
            pltpu.SemaphoreType.DMA,
        ],
    )(_sc_kernel)
    return k(xt, weight)


def _tc2_body(wt_ref, xt_ref, o_ref):
    @pl.when(pl.program_id(1) == 0)
    def _init():
        o_ref[...] = jnp.zeros_like(o_ref)

    o_ref[...] += jax.lax.dot_general(
        wt_ref[...], xt_ref[...],
        dimension_numbers=(((0,), (0,)), ((), ())),
        preferred_element_type=jnp.float32,
    )


def kernel(x, weight):
    B, D = x.shape
    C = weight.shape[0]
    xt = x.T
    tc_out = pl.pallas_call(
        _tc_body,
        grid=(_TC_COLS // _BT,),
        in_specs=[
            pl.BlockSpec((C, D), lambda j: (0, 0)),
            pl.BlockSpec((D, _BT), lambda j: (0, j)),
        ],
        out_specs=pl.BlockSpec((C, _BT), lambda j: (0, j)),
        out_shape=jax.ShapeDtypeStruct((C, _TC_COLS), jnp.float32),
    )(weight, xt)
    rb0 = _SC_D // _RB
    tc2_out = pl.pallas_call(
        _tc2_body,
        grid=(_SC_COLS // _BT, (_D - _SC_D) // _RB),
        in_specs=[
            pl.BlockSpec((_RB, C), lambda j, i: (rb0 + i, 0)),
            pl.BlockSpec((_RB, _BT), lambda j, i: (rb0 + i, _TC_COLS // _BT + j)),
        ],
        out_specs=pl.BlockSpec((C, _BT), lambda j, i: (0, j)),
        out_shape=jax.ShapeDtypeStruct((C, _SC_COLS), jnp.float32),
    )(weight.T, xt)
    sc_out = _sc_matmul(xt, weight)
    out = jnp.concatenate([tc_out, sc_out + tc2_out], axis=1)
    return out.T


# speedup vs baseline: 1.2505x; 1.0441x over previous
import functools
import jax
import jax.numpy as jnp
from jax import lax
from jax.experimental import pallas as pl
from jax.experimental.pallas import tpu as pltpu
from jax.experimental.pallas import tpu_sc as plsc

_BT = 256
_SC_COLS = 4096
_NBW = 128
_DCH = 160
_NG = _NBW // 16
_C = 5
_D = 10000
_TC_COLS = 16384 - _SC_COLS
_SC_D = 3520
_RB = 1000


def _tc_body(w_ref, xt_ref, o_ref):
    o_ref[...] = jax.lax.dot_general(
        w_ref[...], xt_ref[...],
        dimension_numbers=(((1,), (0,)), ((), ())),
        preferred_element_type=jnp.float32,
    )


def _sc_kernel(xt_hbm, w_hbm, o_hbm, wbuf, xbuf0, xbuf1, obuf, sem0, sem1):
    wid = lax.axis_index("s") * 2 + lax.axis_index("c")
    base = wid * _NBW
    src_base = _TC_COLS + base
    n_chunks = _SC_D // _DCH

    pltpu.sync_copy(w_hbm, wbuf)

    def copy_chunk(g, buf, sem):
        return pltpu.make_async_copy(
            xt_hbm.at[pl.ds(g * _DCH, _DCH), pl.ds(src_base, _NBW)], buf, sem
        )

    def consume(g, xbuf, acc):
        d0 = g * _DCH

        def d_body(t, acc):
            wvs = [wbuf[c, pl.ds(d0 + t * 16, 16)] for c in range(_C)]
            out = list(acc)
            for k in range(16):
                ws = [wvs[c][k] for c in range(_C)]
                for gi in range(_NG):
                    xv = xbuf[t * 16 + k, pl.ds(gi * 16, 16)]
                    for c in range(_C):
                        out[gi * _C + c] = out[gi * _C + c] + xv * ws[c]
            return tuple(out)

        return lax.fori_loop(0, _DCH // 16, d_body, acc)

    copy_chunk(0, xbuf0, sem0).start()

    def pair_body(t2, acc):
        g0 = 2 * t2
        copy_chunk(g0 + 1, xbuf1, sem1).start()
        copy_chunk(g0, xbuf0, sem0).wait()
        acc = consume(g0, xbuf0, acc)

        @pl.when(g0 + 2 < n_chunks)
        def _next():
            copy_chunk(g0 + 2, xbuf0, sem0).start()

        copy_chunk(g0 + 1, xbuf1, sem1).wait()
        return consume(g0 + 1, xbuf1, acc)

    acc0 = tuple(jnp.zeros((16,), jnp.float32) for _ in range(_NG * _C))
    acc = lax.fori_loop(0, n_chunks // 2, pair_body, acc0)

    for gi in range(_NG):
        for c in range(_C):
            obuf[c, pl.ds(gi * 16, 16)] = acc[gi * _C + c]
    pltpu.sync_copy(obuf, o_hbm.at[:, pl.ds(base, _NBW)])


def _sc_matmul(xt, weight):
    mesh = plsc.VectorSubcoreMesh(core_axis_name="c", subcore_axis_name="s")
    k = functools.partial(
        pl.kernel,
        out_type=jax.ShapeDtypeStruct((_C, _SC_COLS), jnp.float32),
        mesh=mesh,
        scratch_types=[
            pltpu.VMEM((_C, _D), jnp.float32),
            pltpu.VMEM((_DCH, _NBW), jnp.float32),
            pltpu.VMEM((_DCH, _NBW), jnp.float32),
            pltpu.VMEM((_C, _NBW), jnp.float32),
            pltpu.SemaphoreType.DMA,
            pltpu.SemaphoreType.DMA,
        ],
    )(_sc_kernel)
    return k(xt, weight)


def _tc2_body(wt_ref, xt_ref, o_ref):
    @pl.when(pl.program_id(1) == 0)
    def _init():
        o_ref[...] = jnp.zeros_like(o_ref)

    o_ref[...] += jax.lax.dot_general(
        wt_ref[...], xt_ref[...],
        dimension_numbers=(((0,), (0,)), ((), ())),
        preferred_element_type=jnp.float32,
    )


def kernel(x, weight):
    B, D = x.shape
    C = weight.shape[0]
    xt = x.T
    tc_out = pl.pallas_call(
        _tc_body,
        grid=(_TC_COLS // _BT,),
        in_specs=[
            pl.BlockSpec((C, D), lambda j: (0, 0)),
            pl.BlockSpec((D, _BT), lambda j: (0, j)),
        ],
        out_specs=pl.BlockSpec((C, _BT), lambda j: (0, j)),
        out_shape=jax.ShapeDtypeStruct((C, _TC_COLS), jnp.float32),
    )(weight, xt)
    rb0 = _SC_D // _RB
    wt_tail = jnp.where(jnp.arange(D)[:, None] >= _SC_D, weight.T, 0.0)
    tc2_out = pl.pallas_call(
        _tc2_body,
        grid=(_SC_COLS // _BT, (_D - rb0 * _RB) // _RB),
        in_specs=[
            pl.BlockSpec((_RB, C), lambda j, i: (rb0 + i, 0)),
            pl.BlockSpec((_RB, _BT), lambda j, i: (rb0 + i, _TC_COLS // _BT + j)),
        ],
        out_specs=pl.BlockSpec((C, _BT), lambda j, i: (0, j)),
        out_shape=jax.ShapeDtypeStruct((C, _SC_COLS), jnp.float32),
    )(wt_tail, xt)
    sc_out = _sc_matmul(xt, weight)
    out = jnp.concatenate([tc_out, sc_out + tc2_out], axis=1)
    return out.T


# speedup vs baseline: 1.8822x; 1.5051x over previous
import jax
import jax.numpy as jnp
from jax.experimental import pallas as pl

_BT = 256


def _body(w_ref, xt_ref, o_ref):
    o_ref[...] = jax.lax.dot_general(
        w_ref[...], xt_ref[...],
        dimension_numbers=(((1,), (0,)), ((), ())),
        preferred_element_type=jnp.float32,
    )


def kernel(x, weight):
    B, D = x.shape
    C = weight.shape[0]
    xt = x.T
    out = pl.pallas_call(
        _body,
        grid=(B // _BT,),
        in_specs=[
            pl.BlockSpec((C, D), lambda j: (0, 0)),
            pl.BlockSpec((D, _BT), lambda j: (0, j)),
        ],
        out_specs=pl.BlockSpec((C, _BT), lambda j: (0, j)),
        out_shape=jax.ShapeDtypeStruct((C, B), jnp.float32),
    )(weight, xt)
    return out.T


# speedup vs baseline: 1.8836x; 1.0008x over previous
import jax
import jax.numpy as jnp
from jax.experimental import pallas as pl

_BT = 512


def _body(w_ref, xt_ref, o_ref):
    o_ref[...] = jax.lax.dot_general(
        w_ref[...], xt_ref[...],
        dimension_numbers=(((1,), (0,)), ((), ())),
        preferred_element_type=jnp.float32,
    )


def kernel(x, weight):
    B, D = x.shape
    C = weight.shape[0]
    xt = x.T
    out = pl.pallas_call(
        _body,
        grid=(B // _BT,),
        in_specs=[
            pl.BlockSpec((C, D), lambda j: (0, 0)),
            pl.BlockSpec((D, _BT), lambda j: (0, j)),
        ],
        out_specs=pl.BlockSpec((C, _BT), lambda j: (0, j)),
        out_shape=jax.ShapeDtypeStruct((C, B), jnp.float32),
    )(weight, xt)
    return out.T
